# baseline (speedup 1.0000x reference)
import functools

import jax
import jax.numpy as jnp
from jax import lax
from jax.experimental import pallas as pl
from jax.experimental.pallas import tpu as pltpu
from jax.experimental.pallas import tpu_sc as plsc

N = 10000
E = 320000
H = 128
C = 2
G = 64

NT = 16
NSC = 2
NP = 10240
ROWS_T = NP // NT
EP = 327680
ER = EP // 128
K = 128
GROUP = 10
KR = 2
RB = 1024
NB = NP // RB


def _mesh():
    return plsc.VectorSubcoreMesh(core_axis_name="c", subcore_axis_name="s")


def _deg_body(idx_hbm, out_hbm, idx_v, ones_v, zb_v, acc):
    c = lax.axis_index("c")
    s = lax.axis_index("s")
    w = c * NT + s
    for j in range(128 // 16):
        ones_v[pl.ds(j * 16, 16)] = jnp.full((16,), 1.0, jnp.float32)

    def zf(i, carry):
        zb_v[pl.ds(i * 16, 16)] = jnp.zeros((16,), jnp.float32)
        return carry

    lax.fori_loop(0, ROWS_T // 16, zf, 0)
    pltpu.sync_copy(zb_v, acc.at[pl.ds(s * ROWS_T, ROWS_T)])
    plsc.subcore_barrier()

    rows_per_tile = ER // (NSC * NT)

    def chunk(i, carry):
        r0 = w * rows_per_tile + i * KR
        pltpu.sync_copy(idx_hbm.at[pl.ds(r0, KR), 1, :], idx_v)
        for j in range(KR):
            pltpu.sync_copy(ones_v, acc.at[idx_v.at[j]], add=True)
        return carry

    lax.fori_loop(0, rows_per_tile // KR, chunk, 0)
    plsc.subcore_barrier()
    pltpu.sync_copy(acc.at[pl.ds(s * ROWS_T, ROWS_T)],
                    out_hbm.at[c, pl.ds(s * ROWS_T, ROWS_T)])


_deg_pass = functools.partial(
    pl.kernel,
    out_type=jax.ShapeDtypeStruct((NSC, NP), jnp.float32),
    mesh=_mesh(),
    scratch_types=[
        pltpu.VMEM((KR, 128), jnp.int32),
        pltpu.VMEM((128,), jnp.float32),
        pltpu.VMEM((ROWS_T,), jnp.float32),
        pltpu.VMEM_SHARED((NP,), jnp.float32),
    ],
)(_deg_body)


def _make_apass(class_split, W=H, dt=jnp.float32, do_scatter=True, do_gather=True):
    rows_per_tile = ER // NT if class_split else ER // (NSC * NT)
    ngroups = rows_per_tile // GROUP

    def body(*refs):
        if class_split:
            tab_a, tab_b, idx_hbm, out_hbm = refs[:4]
            scratch = refs[4:]
        else:
            tab_a, idx_hbm, out_hbm = refs[:3]
            scratch = refs[3:]
            tab_b = tab_a
        (ib0, ib1, rows0, rows1, isem0, isem1,
         gsem00, gsem01, gsem10, gsem11, ssem0, ssem1, acc) = scratch
        ibufs, isems = (ib0, ib1), (isem0, isem1)
        rbufs = (rows0, rows1)
        gsems = ((gsem00, gsem01), (gsem10, gsem11))
        ssems = (ssem0, ssem1)
        c = lax.axis_index("c")
        s = lax.axis_index("s")

        def run(tab):
            pltpu.sync_copy(tab.at[pl.ds(s * ROWS_T, ROWS_T), :],
                            acc.at[pl.ds(s * ROWS_T, ROWS_T), :])
            plsc.subcore_barrier()
            if class_split:
                tile_row0 = s * rows_per_tile
            else:
                tile_row0 = (c * NT + s) * rows_per_tile

            def idesc(g, p):
                return pltpu.make_async_copy(
                    idx_hbm.at[pl.ds(tile_row0 + g * GROUP, GROUP), :, :],
                    ibufs[p], isems[p])

            def gfire(ib, j, p):
                return (
                    pltpu.async_copy(tab.at[ib.at[j, 0, pl.ds(0, 64)]],
                                     rbufs[p].at[pl.ds(0, 64), :],
                                     gsems[p][0]),
                    pltpu.async_copy(tab.at[ib.at[j, 0, pl.ds(64, 64)]],
                                     rbufs[p].at[pl.ds(64, 64), :],
                                     gsems[p][1]),
                )

            idesc(0, 0).start()

            def outer(k2, carry):
                for p in range(2):
                    g = 2 * k2 + p
                    idesc(g, p).wait()
                    gnxt = jnp.minimum(g + 1, ngroups - 1)
                    idesc(gnxt, 1 - p).start()
                    ib = ibufs[p]
                    gd = {}
                    if do_gather:
                        gd = {0: gfire(ib, 0, 0), 1: gfire(ib, 1, 1)}
                    sd = {}
                    for j in range(GROUP):
                        b = j % 2
                        for d in gd.get(j, ()):
                            d.wait()
                        if do_scatter:
                            sd[j] = pltpu.async_copy(rbufs[b],
                                                     acc.at[ib.at[j, 1]],
                                                     ssems[b], add=True)
                        if j + 2 < GROUP:
                            if do_scatter:
                                sd[j].wait()
                            if do_gather:
                                gd[j + 2] = gfire(ib, j + 2, b)
                    if do_scatter:
                        sd[GROUP - 2].wait()
                        sd[GROUP - 1].wait()
                return carry

            lax.fori_loop(0, ngroups // 2, outer, 0)
            idesc(ngroups - 1, 0).wait()
            plsc.subcore_barrier()
            pltpu.sync_copy(acc.at[pl.ds(s * ROWS_T, ROWS_T), :],
                            out_hbm.at[c, pl.ds(s * ROWS_T, ROWS_T), :])

        if class_split:
            @pl.when(c == 0)
            def _():
                run(tab_a)

            @pl.when(c != 0)
            def _():
                run(tab_b)
        else:
            run(tab_a)

    return functools.partial(
        pl.kernel,
        out_type=jax.ShapeDtypeStruct((NSC, NP, W), dt),
        mesh=_mesh(),
        scratch_types=[
            pltpu.VMEM((GROUP, 2, 128), jnp.int32),
            pltpu.VMEM((GROUP, 2, 128), jnp.int32),
            pltpu.VMEM((K, W), dt),
            pltpu.VMEM((K, W), dt),
            pltpu.SemaphoreType.DMA,
            pltpu.SemaphoreType.DMA,
            pltpu.SemaphoreType.DMA,
            pltpu.SemaphoreType.DMA,
            pltpu.SemaphoreType.DMA,
            pltpu.SemaphoreType.DMA,
            pltpu.SemaphoreType.DMA,
            pltpu.SemaphoreType.DMA,
            pltpu.VMEM_SHARED((NP, W), dt),
        ],
    )(body)


def _make_apass_pk(class_split):
    rows_per_tile = ER // NT if class_split else ER // (NSC * NT)
    ngroups = rows_per_tile // GROUP

    def body(*refs):
        if class_split:
            tf_a, tf_b, tp_a, tp_b, idx_hbm, out_hbm = refs[:6]
            scratch = refs[6:]
        else:
            tf_a, tp_a, idx_hbm, out_hbm = refs[:4]
            scratch = refs[4:]
            tf_b, tp_b = tf_a, tp_a
        (ib0, ib1, pk0, pk1, fbuf, isem0, isem1,
         gsem0, gsem1, ssem, acc) = scratch
        ibufs, isems = (ib0, ib1), (isem0, isem1)
        pbufs, gsems = (pk0, pk1), (gsem0, gsem1)
        c = lax.axis_index("c")
        s = lax.axis_index("s")

        def cvt(pb):
            @plsc.parallel_loop(0, K, unroll=4)
            def _(r):
                for k in range(4):
                    w = plsc.bitcast(pb[r, pl.ds(16 * k, 16)],
                                     jnp.bfloat16)
                    lo, hi = plsc.unpack(w, format=plsc.PackFormat.INTERLEAVED)
                    fbuf[r, pl.ds(16 * k, 16)] = lo
                    fbuf[r, pl.ds(64 + 16 * k, 16)] = hi

        def run(tf, tp):
            pltpu.sync_copy(tf.at[pl.ds(s * ROWS_T, ROWS_T), :],
                            acc.at[pl.ds(s * ROWS_T, ROWS_T), :])
            plsc.subcore_barrier()
            if class_split:
                tile_row0 = s * rows_per_tile
            else:
                tile_row0 = (c * NT + s) * rows_per_tile

            def idesc(g, p):
                return pltpu.make_async_copy(
                    idx_hbm.at[pl.ds(tile_row0 + g * GROUP, GROUP), :, :],
                    ibufs[p], isems[p])

            def gfire(ib, j, p):
                return pltpu.async_copy(tp.at[ib.at[j, 0]], pbufs[p],
                                        gsems[p])

            idesc(0, 0).start()

            def outer(k2, carry):
                for p in range(2):
                    g = 2 * k2 + p
                    idesc(g, p).wait()
                    gnxt = jnp.minimum(g + 1, ngroups - 1)
                    idesc(gnxt, 1 - p).start()
                    ib = ibufs[p]
                    gd = {0: gfire(ib, 0, 0), 1: gfire(ib, 1, 1)}
                    sd = {}
                    for j in range(GROUP):
                        b = j % 2
                        gd[j].wait()
                        if j >= 1:
                            sd[j - 1].wait()
                        cvt(pbufs[b])
                        if j + 2 < GROUP:
                            gd[j + 2] = gfire(ib, j + 2, b)
                        sd[j] = pltpu.async_copy(fbuf, acc.at[ib.at[j, 1]],
                                                 ssem, add=True)
                    sd[GROUP - 1].wait()
                return carry

            lax.fori_loop(0, ngroups // 2, outer, 0)
            idesc(ngroups - 1, 0).wait()
            plsc.subcore_barrier()
            pltpu.sync_copy(acc.at[pl.ds(s * ROWS_T, ROWS_T), :],
                            out_hbm.at[c, pl.ds(s * ROWS_T, ROWS_T), :])

        if class_split:
            @pl.when(c == 0)
            def _():
                run(tf_a, tp_a)

            @pl.when(c != 0)
            def _():
                run(tf_b, tp_b)
        else:
            run(tf_a, tp_a)

    return functools.partial(
        pl.kernel,
        out_type=jax.ShapeDtypeStruct((NSC, NP, H), jnp.float32),
        mesh=_mesh(),
        compiler_params=pltpu.CompilerParams(use_tc_tiling_on_sc=False,
                                             needs_layout_passes=False),
        scratch_types=[
            pltpu.VMEM((GROUP, 2, 128), jnp.int32),
            pltpu.VMEM((GROUP, 2, 128), jnp.int32),
            pltpu.VMEM((K, H // 2), jnp.int32),
            pltpu.VMEM((K, H // 2), jnp.int32),
            pltpu.VMEM((K, H), jnp.float32),
            pltpu.SemaphoreType.DMA,
            pltpu.SemaphoreType.DMA,
            pltpu.SemaphoreType.DMA,
            pltpu.SemaphoreType.DMA,
            pltpu.SemaphoreType.DMA,
            pltpu.VMEM_SHARED((NP, H), jnp.float32),
        ],
    )(body)


def _pack_tab(sf):
    sb = sf.astype(jnp.bfloat16)
    pk = jnp.stack([sb[..., :64], sb[..., 64:]], axis=-1)
    return jax.lax.bitcast_convert_type(pk, jnp.int32)


_apass_shared_pk = _make_apass_pk(class_split=False)
_apass_class_pk = _make_apass_pk(class_split=True)

_apass_shared = _make_apass(class_split=False)
_apass_class = _make_apass(class_split=True)


def _prescale_body(x_ref, d0_ref, d1_ref, dinv_ref, s0_ref):
    deg = d0_ref[...] + d1_ref[...] + 1.0
    dinv = 1.0 / jnp.sqrt(deg)
    dinv_ref[...] = dinv
    s0_ref[...] = x_ref[...] * dinv


def _prescale(xp, d0, d1):
    return pl.pallas_call(
        _prescale_body,
        grid=(NB,),
        in_specs=[
            pl.BlockSpec((RB, H), lambda i: (i, 0)),
            pl.BlockSpec((RB, 1), lambda i: (i, 0)),
            pl.BlockSpec((RB, 1), lambda i: (i, 0)),
        ],
        out_specs=[
            pl.BlockSpec((RB, 1), lambda i: (i, 0)),
            pl.BlockSpec((RB, H), lambda i: (i, 0)),
        ],
        out_shape=[
            jax.ShapeDtypeStruct((NP, 1), jnp.float32),
            jax.ShapeDtypeStruct((NP, H), jnp.float32),
        ],
    )(xp, d0, d1)


def _l2relu(q):
    r2 = jnp.sum(q * q, axis=1, keepdims=True)
    nrm = jnp.maximum(jnp.sqrt(r2), 1e-12)
    return jnp.maximum(q / nrm, 0.0)


def _dense1_body(ua_ref, ub_ref, s0_ref, dinv_ref, w_ref, b_ref, out_ref):
    dinv = dinv_ref[...]
    t = (ua_ref[0] + ub_ref[0] - s0_ref[...]) * dinv
    q = jnp.dot(t, w_ref[0], preferred_element_type=jnp.float32) + b_ref[0]
    out_ref[0] = _l2relu(q) * dinv


def _dense1(u1p, s0, dinv, w, b):
    return pl.pallas_call(
        _dense1_body,
        grid=(C, NB),
        in_specs=[
            pl.BlockSpec((1, RB, H), lambda c, i: (0, i, 0)),
            pl.BlockSpec((1, RB, H), lambda c, i: (1, i, 0)),
            pl.BlockSpec((RB, H), lambda c, i: (i, 0)),
            pl.BlockSpec((RB, 1), lambda c, i: (i, 0)),
            pl.BlockSpec((1, H, H), lambda c, i: (c, 0, 0)),
            pl.BlockSpec((1, 1, H), lambda c, i: (c, 0, 0)),
        ],
        out_specs=pl.BlockSpec((1, RB, H), lambda c, i: (c, i, 0)),
        out_shape=jax.ShapeDtypeStruct((C, NP, H), jnp.float32),
    )(u1p, u1p, s0, dinv, w, b)


def _make_dense23(prescale_out):
    def body(u_ref, dinv_ref, w_ref, b_ref, out_ref):
        dinv = dinv_ref[...]
        t = u_ref[0] * dinv
        q = jnp.dot(t, w_ref[0], preferred_element_type=jnp.float32) + b_ref[0]
        h = _l2relu(q)
        out_ref[0] = h * dinv if prescale_out else h

    def call(u, dinv, w, b):
        return pl.pallas_call(
            body,
            grid=(C, NB),
            in_specs=[
                pl.BlockSpec((1, RB, H), lambda c, i: (c, i, 0)),
                pl.BlockSpec((RB, 1), lambda c, i: (i, 0)),
                pl.BlockSpec((1, H, H), lambda c, i: (c, 0, 0)),
                pl.BlockSpec((1, 1, H), lambda c, i: (c, 0, 0)),
            ],
            out_specs=pl.BlockSpec((1, RB, H), lambda c, i: (c, i, 0)),
            out_shape=jax.ShapeDtypeStruct((C, NP, H), jnp.float32),
        )(u, dinv, w, b)

    return call


_dense2 = _make_dense23(prescale_out=True)
_dense3 = _make_dense23(prescale_out=False)


def _pool_body(h_ref, batch_ref, w1_ref, b1_ref, w2_ref, b2_ref, out_ref):
    hb = h_ref[0]
    bt = batch_ref[0]
    gids = lax.broadcasted_iota(jnp.int32, (G, NP), 0)
    oh = (bt == gids).astype(jnp.float32)
    pooled = jnp.dot(oh, hb, preferred_element_type=jnp.float32)
    z = jnp.dot(pooled, w1_ref[0], preferred_element_type=jnp.float32)
    z = jnp.maximum(z + b1_ref[0], 0.0)
    o = jnp.sum(z * w2_ref[0], axis=1) + b2_ref[0, 0]
    out_ref[0, 0] = o


def _pool_head(h3, batchp, w1, b1, w2t, b2):
    return pl.pallas_call(
        _pool_body,
        grid=(C,),
        in_specs=[
            pl.BlockSpec((1, NP, H), lambda c: (c, 0, 0)),
            pl.BlockSpec((1, 1, NP), lambda c: (0, 0, 0)),
            pl.BlockSpec((1, H, H), lambda c: (c, 0, 0)),
            pl.BlockSpec((1, 1, H), lambda c: (c, 0, 0)),
            pl.BlockSpec((1, 1, H), lambda c: (c, 0, 0)),
            pl.BlockSpec((1, 1, 1), lambda c: (c, 0, 0)),
        ],
        out_specs=pl.BlockSpec((1, 1, G), lambda c: (c, 0, 0)),
        out_shape=jax.ShapeDtypeStruct((C, 1, G), jnp.float32),
    )(h3, batchp, w1, b1, w2t, b2)


def kernel(x, edge_index, batch, conv_W0, conv_b0, conv_W1, conv_b1,
           conv_W2, conv_b2, lin1_W, lin1_b, lin2_W, lin2_b):
    pad_e = EP - E
    srcp = jnp.concatenate(
        [edge_index[0], jnp.zeros((pad_e,), edge_index.dtype)]).reshape(ER, 128)
    dstp = jnp.concatenate(
        [edge_index[1], jnp.full((pad_e,), N, edge_index.dtype)]).reshape(ER, 128)
    idx2 = jnp.stack([srcp, dstp], axis=1)
    xp = jnp.pad(x, ((0, NP - N), (0, 0)))
    batchp = jnp.pad(batch, (0, NP - N), constant_values=G).reshape(1, 1, NP)

    degp = _deg_pass(idx2)
    d0 = degp[0].reshape(NP, 1)
    d1 = degp[1].reshape(NP, 1)
    dinv, s0 = _prescale(xp, d0, d1)

    b0 = conv_b0.reshape(C, 1, H)
    b1 = conv_b1.reshape(C, 1, H)
    b2 = conv_b2.reshape(C, 1, H)
    l1b = lin1_b.reshape(C, 1, H)
    w2t = jnp.transpose(lin2_W, (0, 2, 1))
    l2b = lin2_b.reshape(C, 1, 1)

    u1p = _apass_shared_pk(s0, _pack_tab(s0), idx2)
    s1 = _dense1(u1p, s0, dinv, conv_W0, b0)
    s1p = _pack_tab(s1)
    u2 = _apass_class_pk(s1[0], s1[1], s1p[0], s1p[1], idx2)
    s2 = _dense2(u2, dinv, conv_W1, b1)
    u3 = _apass_class(s2[0], s2[1], idx2)
    h3 = _dense3(u3, dinv, conv_W2, b2)
    out = _pool_head(h3, batchp, lin1_W, l1b, w2t, l2b)
    return jnp.transpose(out[:, 0, :], (1, 0))


# speedup vs baseline: 1.0002x; 1.0002x over previous
import functools

import jax
import jax.numpy as jnp
from jax import lax
from jax.experimental import pallas as pl
from jax.experimental.pallas import tpu as pltpu
from jax.experimental.pallas import tpu_sc as plsc

N = 10000
E = 320000
H = 128
C = 2
G = 64

NT = 16
NSC = 2
NP = 10240
ROWS_T = NP // NT
EP = 327680
ER = EP // 128
K = 128
GROUP = 10
KR = 2
RB = 1024
NB = NP // RB


def _mesh():
    return plsc.VectorSubcoreMesh(core_axis_name="c", subcore_axis_name="s")


def _deg_body(idx_hbm, out_hbm, idx_v, ones_v, zb_v, acc):
    c = lax.axis_index("c")
    s = lax.axis_index("s")
    w = c * NT + s
    for j in range(128 // 16):
        ones_v[pl.ds(j * 16, 16)] = jnp.full((16,), 1.0, jnp.float32)

    def zf(i, carry):
        zb_v[pl.ds(i * 16, 16)] = jnp.zeros((16,), jnp.float32)
        return carry

    lax.fori_loop(0, ROWS_T // 16, zf, 0)
    pltpu.sync_copy(zb_v, acc.at[pl.ds(s * ROWS_T, ROWS_T)])
    plsc.subcore_barrier()

    rows_per_tile = ER // (NSC * NT)

    def chunk(i, carry):
        r0 = w * rows_per_tile + i * KR
        pltpu.sync_copy(idx_hbm.at[pl.ds(r0, KR), 1, :], idx_v)
        for j in range(KR):
            pltpu.sync_copy(ones_v, acc.at[idx_v.at[j]], add=True)
        return carry

    lax.fori_loop(0, rows_per_tile // KR, chunk, 0)
    plsc.subcore_barrier()
    pltpu.sync_copy(acc.at[pl.ds(s * ROWS_T, ROWS_T)],
                    out_hbm.at[c, pl.ds(s * ROWS_T, ROWS_T)])


_deg_pass = functools.partial(
    pl.kernel,
    out_type=jax.ShapeDtypeStruct((NSC, NP), jnp.float32),
    mesh=_mesh(),
    scratch_types=[
        pltpu.VMEM((KR, 128), jnp.int32),
        pltpu.VMEM((128,), jnp.float32),
        pltpu.VMEM((ROWS_T,), jnp.float32),
        pltpu.VMEM_SHARED((NP,), jnp.float32),
    ],
)(_deg_body)


def _make_apass(class_split):
    rows_per_tile = ER // NT if class_split else ER // (NSC * NT)
    ngroups = rows_per_tile // GROUP

    def body(*refs):
        if class_split:
            tab_a, tab_b, idx_hbm, out_hbm = refs[:4]
            scratch = refs[4:]
        else:
            tab_a, idx_hbm, out_hbm = refs[:3]
            scratch = refs[3:]
            tab_b = tab_a
        (ib0, ib1, rows0, rows1, isem0, isem1,
         gsem00, gsem01, gsem10, gsem11, ssem0, ssem1, acc) = scratch
        ibufs, isems = (ib0, ib1), (isem0, isem1)
        rbufs = (rows0, rows1)
        gsems = ((gsem00, gsem01), (gsem10, gsem11))
        ssems = (ssem0, ssem1)
        c = lax.axis_index("c")
        s = lax.axis_index("s")

        def run(tab):
            pltpu.sync_copy(tab.at[pl.ds(s * ROWS_T, ROWS_T), :],
                            acc.at[pl.ds(s * ROWS_T, ROWS_T), :])
            plsc.subcore_barrier()
            if class_split:
                tile_row0 = s * rows_per_tile
            else:
                tile_row0 = (c * NT + s) * rows_per_tile

            def idesc(g, p):
                return pltpu.make_async_copy(
                    idx_hbm.at[pl.ds(tile_row0 + g * GROUP, GROUP), :, :],
                    ibufs[p], isems[p])

            def gfire(ib, j, p):
                return (
                    pltpu.async_copy(tab.at[ib.at[j, 0, pl.ds(0, 64)]],
                                     rbufs[p].at[pl.ds(0, 64), :],
                                     gsems[p][0]),
                    pltpu.async_copy(tab.at[ib.at[j, 0, pl.ds(64, 64)]],
                                     rbufs[p].at[pl.ds(64, 64), :],
                                     gsems[p][1]),
                )

            idesc(0, 0).start()

            def outer(k2, carry):
                for p in range(2):
                    g = 2 * k2 + p
                    idesc(g, p).wait()
                    gnxt = jnp.minimum(g + 1, ngroups - 1)
                    idesc(gnxt, 1 - p).start()
                    ib = ibufs[p]
                    gd = {0: gfire(ib, 0, 0), 1: gfire(ib, 1, 1)}
                    sd = {}
                    for j in range(GROUP):
                        b = j % 2
                        for d in gd[j]:
                            d.wait()
                        sd[j] = pltpu.async_copy(rbufs[b],
                                                 acc.at[ib.at[j, 1]],
                                                 ssems[b], add=True)
                        if j + 2 < GROUP:
                            sd[j].wait()
                            gd[j + 2] = gfire(ib, j + 2, b)
                    sd[GROUP - 2].wait()
                    sd[GROUP - 1].wait()
                return carry

            lax.fori_loop(0, ngroups // 2, outer, 0)
            idesc(ngroups - 1, 0).wait()
            plsc.subcore_barrier()
            pltpu.sync_copy(acc.at[pl.ds(s * ROWS_T, ROWS_T), :],
                            out_hbm.at[c, pl.ds(s * ROWS_T, ROWS_T), :])

        if class_split:
            @pl.when(c == 0)
            def _():
                run(tab_a)

            @pl.when(c != 0)
            def _():
                run(tab_b)
        else:
            run(tab_a)

    return functools.partial(
        pl.kernel,
        out_type=jax.ShapeDtypeStruct((NSC, NP, H), jnp.float32),
        mesh=_mesh(),
        scratch_types=[
            pltpu.VMEM((GROUP, 2, 128), jnp.int32),
            pltpu.VMEM((GROUP, 2, 128), jnp.int32),
            pltpu.VMEM((K, H), jnp.float32),
            pltpu.VMEM((K, H), jnp.float32),
            pltpu.SemaphoreType.DMA,
            pltpu.SemaphoreType.DMA,
            pltpu.SemaphoreType.DMA,
            pltpu.SemaphoreType.DMA,
            pltpu.SemaphoreType.DMA,
            pltpu.SemaphoreType.DMA,
            pltpu.SemaphoreType.DMA,
            pltpu.SemaphoreType.DMA,
            pltpu.VMEM_SHARED((NP, H), jnp.float32),
        ],
    )(body)


def _make_apass_pk(class_split):
    rows_per_tile = ER // NT if class_split else ER // (NSC * NT)
    ngroups = rows_per_tile // GROUP

    def body(*refs):
        if class_split:
            tf_a, tf_b, tp_a, tp_b, idx_hbm, out_hbm = refs[:6]
            scratch = refs[6:]
        else:
            tf_a, tp_a, idx_hbm, out_hbm = refs[:4]
            scratch = refs[4:]
            tf_b, tp_b = tf_a, tp_a
        (ib0, ib1, pk0, pk1, fbuf, isem0, isem1,
         gsem0, gsem1, ssem, acc) = scratch
        ibufs, isems = (ib0, ib1), (isem0, isem1)
        pbufs, gsems = (pk0, pk1), (gsem0, gsem1)
        c = lax.axis_index("c")
        s = lax.axis_index("s")

        def cvt(pb):
            @plsc.parallel_loop(0, K, unroll=4)
            def _(r):
                for k in range(4):
                    w = plsc.bitcast(pb[r, pl.ds(16 * k, 16)],
                                     jnp.bfloat16)
                    lo, hi = plsc.unpack(w, format=plsc.PackFormat.INTERLEAVED)
                    fbuf[r, pl.ds(16 * k, 16)] = lo
                    fbuf[r, pl.ds(64 + 16 * k, 16)] = hi

        def run(tf, tp):
            pltpu.sync_copy(tf.at[pl.ds(s * ROWS_T, ROWS_T), :],
                            acc.at[pl.ds(s * ROWS_T, ROWS_T), :])
            plsc.subcore_barrier()
            if class_split:
                tile_row0 = s * rows_per_tile
            else:
                tile_row0 = (c * NT + s) * rows_per_tile

            def idesc(g, p):
                return pltpu.make_async_copy(
                    idx_hbm.at[pl.ds(tile_row0 + g * GROUP, GROUP), :, :],
                    ibufs[p], isems[p])

            def gfire(ib, j, p):
                return pltpu.async_copy(tp.at[ib.at[j, 0]], pbufs[p],
                                        gsems[p])

            idesc(0, 0).start()

            def outer(k2, carry):
                for p in range(2):
                    g = 2 * k2 + p
                    idesc(g, p).wait()
                    gnxt = jnp.minimum(g + 1, ngroups - 1)
                    idesc(gnxt, 1 - p).start()
                    ib = ibufs[p]
                    gd = {0: gfire(ib, 0, 0), 1: gfire(ib, 1, 1)}
                    sd = {}
                    for j in range(GROUP):
                        b = j % 2
                        gd[j].wait()
                        if j >= 1:
                            sd[j - 1].wait()
                        cvt(pbufs[b])
                        if j + 2 < GROUP:
                            gd[j + 2] = gfire(ib, j + 2, b)
                        sd[j] = pltpu.async_copy(fbuf, acc.at[ib.at[j, 1]],
                                                 ssem, add=True)
                    sd[GROUP - 1].wait()
                return carry

            lax.fori_loop(0, ngroups // 2, outer, 0)
            idesc(ngroups - 1, 0).wait()
            plsc.subcore_barrier()
            pltpu.sync_copy(acc.at[pl.ds(s * ROWS_T, ROWS_T), :],
                            out_hbm.at[c, pl.ds(s * ROWS_T, ROWS_T), :])

        if class_split:
            @pl.when(c == 0)
            def _():
                run(tf_a, tp_a)

            @pl.when(c != 0)
            def _():
                run(tf_b, tp_b)
        else:
            run(tf_a, tp_a)

    return functools.partial(
        pl.kernel,
        out_type=jax.ShapeDtypeStruct((NSC, NP, H), jnp.float32),
        mesh=_mesh(),
        compiler_params=pltpu.CompilerParams(use_tc_tiling_on_sc=False,
                                             needs_layout_passes=False),
        scratch_types=[
            pltpu.VMEM((GROUP, 2, 128), jnp.int32),
            pltpu.VMEM((GROUP, 2, 128), jnp.int32),
            pltpu.VMEM((K, H // 2), jnp.int32),
            pltpu.VMEM((K, H // 2), jnp.int32),
            pltpu.VMEM((K, H), jnp.float32),
            pltpu.SemaphoreType.DMA,
            pltpu.SemaphoreType.DMA,
            pltpu.SemaphoreType.DMA,
            pltpu.SemaphoreType.DMA,
            pltpu.SemaphoreType.DMA,
            pltpu.VMEM_SHARED((NP, H), jnp.float32),
        ],
    )(body)


def _pack_tab(sf):
    sb = sf.astype(jnp.bfloat16)
    pk = jnp.stack([sb[..., :64], sb[..., 64:]], axis=-1)
    return jax.lax.bitcast_convert_type(pk, jnp.int32)


_apass_shared_pk = _make_apass_pk(class_split=False)
_apass_class_pk = _make_apass_pk(class_split=True)

_apass_class = _make_apass(class_split=True)


def _prescale_body(x_ref, d0_ref, d1_ref, dinv_ref, s0_ref):
    deg = d0_ref[...] + d1_ref[...] + 1.0
    dinv = 1.0 / jnp.sqrt(deg)
    dinv_ref[...] = dinv
    s0_ref[...] = x_ref[...] * dinv


def _prescale(xp, d0, d1):
    return pl.pallas_call(
        _prescale_body,
        grid=(NB,),
        in_specs=[
            pl.BlockSpec((RB, H), lambda i: (i, 0)),
            pl.BlockSpec((RB, 1), lambda i: (i, 0)),
            pl.BlockSpec((RB, 1), lambda i: (i, 0)),
        ],
        out_specs=[
            pl.BlockSpec((RB, 1), lambda i: (i, 0)),
            pl.BlockSpec((RB, H), lambda i: (i, 0)),
        ],
        out_shape=[
            jax.ShapeDtypeStruct((NP, 1), jnp.float32),
            jax.ShapeDtypeStruct((NP, H), jnp.float32),
        ],
    )(xp, d0, d1)


def _l2relu(q):
    r2 = jnp.sum(q * q, axis=1, keepdims=True)
    nrm = jnp.maximum(jnp.sqrt(r2), 1e-12)
    return jnp.maximum(q / nrm, 0.0)


def _dense1_body(ua_ref, ub_ref, s0_ref, dinv_ref, w_ref, b_ref, out_ref):
    dinv = dinv_ref[...]
    t = (ua_ref[0] + ub_ref[0] - s0_ref[...]) * dinv
    q = jnp.dot(t, w_ref[0], preferred_element_type=jnp.float32) + b_ref[0]
    out_ref[0] = _l2relu(q) * dinv


def _dense1(u1p, s0, dinv, w, b):
    return pl.pallas_call(
        _dense1_body,
        grid=(C, NB),
        in_specs=[
            pl.BlockSpec((1, RB, H), lambda c, i: (0, i, 0)),
            pl.BlockSpec((1, RB, H), lambda c, i: (1, i, 0)),
            pl.BlockSpec((RB, H), lambda c, i: (i, 0)),
            pl.BlockSpec((RB, 1), lambda c, i: (i, 0)),
            pl.BlockSpec((1, H, H), lambda c, i: (c, 0, 0)),
            pl.BlockSpec((1, 1, H), lambda c, i: (c, 0, 0)),
        ],
        out_specs=pl.BlockSpec((1, RB, H), lambda c, i: (c, i, 0)),
        out_shape=jax.ShapeDtypeStruct((C, NP, H), jnp.float32),
    )(u1p, u1p, s0, dinv, w, b)


def _make_dense23(prescale_out):
    def body(u_ref, dinv_ref, w_ref, b_ref, out_ref):
        dinv = dinv_ref[...]
        t = u_ref[0] * dinv
        q = jnp.dot(t, w_ref[0], preferred_element_type=jnp.float32) + b_ref[0]
        h = _l2relu(q)
        out_ref[0] = h * dinv if prescale_out else h

    def call(u, dinv, w, b):
        return pl.pallas_call(
            body,
            grid=(C, NB),
            in_specs=[
                pl.BlockSpec((1, RB, H), lambda c, i: (c, i, 0)),
                pl.BlockSpec((RB, 1), lambda c, i: (i, 0)),
                pl.BlockSpec((1, H, H), lambda c, i: (c, 0, 0)),
                pl.BlockSpec((1, 1, H), lambda c, i: (c, 0, 0)),
            ],
            out_specs=pl.BlockSpec((1, RB, H), lambda c, i: (c, i, 0)),
            out_shape=jax.ShapeDtypeStruct((C, NP, H), jnp.float32),
        )(u, dinv, w, b)

    return call


_dense2 = _make_dense23(prescale_out=True)
_dense3 = _make_dense23(prescale_out=False)


def _pool_body(h_ref, batch_ref, w1_ref, b1_ref, w2_ref, b2_ref, out_ref):
    hb = h_ref[0]
    bt = batch_ref[0]
    gids = lax.broadcasted_iota(jnp.int32, (G, NP), 0)
    oh = (bt == gids).astype(jnp.float32)
    pooled = jnp.dot(oh, hb, preferred_element_type=jnp.float32)
    z = jnp.dot(pooled, w1_ref[0], preferred_element_type=jnp.float32)
    z = jnp.maximum(z + b1_ref[0], 0.0)
    o = jnp.sum(z * w2_ref[0], axis=1) + b2_ref[0, 0]
    out_ref[0, 0] = o


def _pool_head(h3, batchp, w1, b1, w2t, b2):
    return pl.pallas_call(
        _pool_body,
        grid=(C,),
        in_specs=[
            pl.BlockSpec((1, NP, H), lambda c: (c, 0, 0)),
            pl.BlockSpec((1, 1, NP), lambda c: (0, 0, 0)),
            pl.BlockSpec((1, H, H), lambda c: (c, 0, 0)),
            pl.BlockSpec((1, 1, H), lambda c: (c, 0, 0)),
            pl.BlockSpec((1, 1, H), lambda c: (c, 0, 0)),
            pl.BlockSpec((1, 1, 1), lambda c: (c, 0, 0)),
        ],
        out_specs=pl.BlockSpec((1, 1, G), lambda c: (c, 0, 0)),
        out_shape=jax.ShapeDtypeStruct((C, 1, G), jnp.float32),
    )(h3, batchp, w1, b1, w2t, b2)


def kernel(x, edge_index, batch, conv_W0, conv_b0, conv_W1, conv_b1,
           conv_W2, conv_b2, lin1_W, lin1_b, lin2_W, lin2_b):
    pad_e = EP - E
    srcp = jnp.concatenate(
        [edge_index[0], jnp.zeros((pad_e,), edge_index.dtype)]).reshape(ER, 128)
    dstp = jnp.concatenate(
        [edge_index[1], jnp.full((pad_e,), N, edge_index.dtype)]).reshape(ER, 128)
    idx2 = jnp.stack([srcp, dstp], axis=1)
    xp = jnp.pad(x, ((0, NP - N), (0, 0)))
    batchp = jnp.pad(batch, (0, NP - N), constant_values=G).reshape(1, 1, NP)

    degp = _deg_pass(idx2)
    d0 = degp[0].reshape(NP, 1)
    d1 = degp[1].reshape(NP, 1)
    dinv, s0 = _prescale(xp, d0, d1)

    b0 = conv_b0.reshape(C, 1, H)
    b1 = conv_b1.reshape(C, 1, H)
    b2 = conv_b2.reshape(C, 1, H)
    l1b = lin1_b.reshape(C, 1, H)
    w2t = jnp.transpose(lin2_W, (0, 2, 1))
    l2b = lin2_b.reshape(C, 1, 1)

    u1p = _apass_shared_pk(s0, _pack_tab(s0), idx2)
    s1 = _dense1(u1p, s0, dinv, conv_W0, b0)
    s1p = _pack_tab(s1)
    u2 = _apass_class_pk(s1[0], s1[1], s1p[0], s1p[1], idx2)
    s2 = _dense2(u2, dinv, conv_W1, b1)
    u3 = _apass_class(s2[0], s2[1], idx2)
    h3 = _dense3(u3, dinv, conv_W2, b2)
    out = _pool_head(h3, batchp, lin1_W, l1b, w2t, l2b)
    return jnp.transpose(out[:, 0, :], (1, 0))


# speedup vs baseline: 1.1742x; 1.1740x over previous
import functools

import jax
import jax.numpy as jnp
from jax import lax
from jax.experimental import pallas as pl
from jax.experimental.pallas import tpu as pltpu
from jax.experimental.pallas import tpu_sc as plsc

N = 10000
E = 320000
H = 128
C = 2
G = 64

NT = 16
NSC = 2
NP = 10240
ROWS_T = NP // NT
EP = 327680
ER = EP // 128
K = 128
GROUP = 10
KR = 2
RB = 1024
NB = NP // RB


def _mesh():
    return plsc.VectorSubcoreMesh(core_axis_name="c", subcore_axis_name="s")


def _deg_body(idx_hbm, out_hbm, idx_v, ones_v, zb_v, acc):
    c = lax.axis_index("c")
    s = lax.axis_index("s")
    w = c * NT + s
    for j in range(128 // 16):
        ones_v[pl.ds(j * 16, 16)] = jnp.full((16,), 1.0, jnp.float32)

    def zf(i, carry):
        zb_v[pl.ds(i * 16, 16)] = jnp.zeros((16,), jnp.float32)
        return carry

    lax.fori_loop(0, ROWS_T // 16, zf, 0)
    pltpu.sync_copy(zb_v, acc.at[pl.ds(s * ROWS_T, ROWS_T)])
    plsc.subcore_barrier()

    rows_per_tile = ER // (NSC * NT)

    def chunk(i, carry):
        r0 = w * rows_per_tile + i * KR
        pltpu.sync_copy(idx_hbm.at[pl.ds(r0, KR), 1, :], idx_v)
        for j in range(KR):
            pltpu.sync_copy(ones_v, acc.at[idx_v.at[j]], add=True)
        return carry

    lax.fori_loop(0, rows_per_tile // KR, chunk, 0)
    plsc.subcore_barrier()
    pltpu.sync_copy(acc.at[pl.ds(s * ROWS_T, ROWS_T)],
                    out_hbm.at[c, pl.ds(s * ROWS_T, ROWS_T)])


_deg_pass = functools.partial(
    pl.kernel,
    out_type=jax.ShapeDtypeStruct((NSC, NP), jnp.float32),
    mesh=_mesh(),
    scratch_types=[
        pltpu.VMEM((KR, 128), jnp.int32),
        pltpu.VMEM((128,), jnp.float32),
        pltpu.VMEM((ROWS_T,), jnp.float32),
        pltpu.VMEM_SHARED((NP,), jnp.float32),
    ],
)(_deg_body)


def _make_apass(class_split):
    rows_per_tile = ER // NT if class_split else ER // (NSC * NT)
    ngroups = rows_per_tile // GROUP

    def body(*refs):
        if class_split:
            tab_a, tab_b, idx_hbm, out_hbm = refs[:4]
            scratch = refs[4:]
        else:
            tab_a, idx_hbm, out_hbm = refs[:3]
            scratch = refs[3:]
            tab_b = tab_a
        (ib0, ib1, rows0, rows1, isem0, isem1,
         gsem00, gsem01, gsem10, gsem11, ssem0, ssem1, acc) = scratch
        ibufs, isems = (ib0, ib1), (isem0, isem1)
        rbufs = (rows0, rows1)
        gsems = ((gsem00, gsem01), (gsem10, gsem11))
        ssems = (ssem0, ssem1)
        c = lax.axis_index("c")
        s = lax.axis_index("s")

        def run(tab):
            pltpu.sync_copy(tab.at[pl.ds(s * ROWS_T, ROWS_T), :],
                            acc.at[pl.ds(s * ROWS_T, ROWS_T), :])
            plsc.subcore_barrier()
            if class_split:
                tile_row0 = s * rows_per_tile
            else:
                tile_row0 = (c * NT + s) * rows_per_tile

            def idesc(g, p):
                return pltpu.make_async_copy(
                    idx_hbm.at[pl.ds(tile_row0 + g * GROUP, GROUP), :, :],
                    ibufs[p], isems[p])

            def gfire(ib, j, p):
                return (
                    pltpu.async_copy(tab.at[ib.at[j, 0, pl.ds(0, 64)]],
                                     rbufs[p].at[pl.ds(0, 64), :],
                                     gsems[p][0]),
                    pltpu.async_copy(tab.at[ib.at[j, 0, pl.ds(64, 64)]],
                                     rbufs[p].at[pl.ds(64, 64), :],
                                     gsems[p][1]),
                )

            idesc(0, 0).start()

            def outer(k2, carry):
                for p in range(2):
                    g = 2 * k2 + p
                    idesc(g, p).wait()
                    gnxt = jnp.minimum(g + 1, ngroups - 1)
                    idesc(gnxt, 1 - p).start()
                    ib = ibufs[p]
                    gd = {0: gfire(ib, 0, 0), 1: gfire(ib, 1, 1)}
                    sd = {}
                    for j in range(GROUP):
                        b = j % 2
                        for d in gd[j]:
                            d.wait()
                        sd[j] = pltpu.async_copy(rbufs[b],
                                                 acc.at[ib.at[j, 1]],
                                                 ssems[b], add=True)
                        if j + 2 < GROUP:
                            sd[j].wait()
                            gd[j + 2] = gfire(ib, j + 2, b)
                    sd[GROUP - 2].wait()
                    sd[GROUP - 1].wait()
                return carry

            lax.fori_loop(0, ngroups // 2, outer, 0)
            idesc(ngroups - 1, 0).wait()
            plsc.subcore_barrier()
            pltpu.sync_copy(acc.at[pl.ds(s * ROWS_T, ROWS_T), :],
                            out_hbm.at[c, pl.ds(s * ROWS_T, ROWS_T), :])

        if class_split:
            @pl.when(c == 0)
            def _():
                run(tab_a)

            @pl.when(c != 0)
            def _():
                run(tab_b)
        else:
            run(tab_a)

    return functools.partial(
        pl.kernel,
        out_type=jax.ShapeDtypeStruct((NSC, NP, H), jnp.float32),
        mesh=_mesh(),
        scratch_types=[
            pltpu.VMEM((GROUP, 2, 128), jnp.int32),
            pltpu.VMEM((GROUP, 2, 128), jnp.int32),
            pltpu.VMEM((K, H), jnp.float32),
            pltpu.VMEM((K, H), jnp.float32),
            pltpu.SemaphoreType.DMA,
            pltpu.SemaphoreType.DMA,
            pltpu.SemaphoreType.DMA,
            pltpu.SemaphoreType.DMA,
            pltpu.SemaphoreType.DMA,
            pltpu.SemaphoreType.DMA,
            pltpu.SemaphoreType.DMA,
            pltpu.SemaphoreType.DMA,
            pltpu.VMEM_SHARED((NP, H), jnp.float32),
        ],
    )(body)


def _make_apass_pk(class_split, unpack_kind='bf16'):
    rows_per_tile = ER // NT if class_split else ER // (NSC * NT)
    ngroups = rows_per_tile // GROUP

    def body(*refs):
        if class_split:
            tf_a, tf_b, tp_a, tp_b, idx_hbm, out_hbm = refs[:6]
            scratch = refs[6:]
        else:
            tf_a, tp_a, idx_hbm, out_hbm = refs[:4]
            scratch = refs[4:]
            tf_b, tp_b = tf_a, tp_a
        (ib0, ib1, pk0, pk1, fbuf, isem0, isem1,
         gsem0, gsem1, ssem, acc) = scratch
        ibufs, isems = (ib0, ib1), (isem0, isem1)
        pbufs, gsems = (pk0, pk1), (gsem0, gsem1)
        c = lax.axis_index("c")
        s = lax.axis_index("s")

        def cvt(pb):
            if unpack_kind == 'bf16':
                @plsc.parallel_loop(0, K, unroll=4)
                def _(r):
                    for k in range(4):
                        w = plsc.bitcast(pb[r, pl.ds(16 * k, 16)],
                                         jnp.bfloat16)
                        lo, hi = plsc.unpack(w, format=plsc.PackFormat.INTERLEAVED)
                        fbuf[r, pl.ds(16 * k, 16)] = lo
                        fbuf[r, pl.ds(64 + 16 * k, 16)] = hi
            else:
                inv = jnp.float32(1.0 / 65535.0)

                @plsc.parallel_loop(0, K, unroll=4)
                def _(r):
                    for k in range(4):
                        w = pb[r, pl.ds(16 * k, 16)]
                        lo = (w & 0xFFFF).astype(jnp.float32) * inv
                        hi = ((w >> 16) & 0xFFFF).astype(jnp.float32) * inv
                        fbuf[r, pl.ds(16 * k, 16)] = lo
                        fbuf[r, pl.ds(64 + 16 * k, 16)] = hi

        def run(tf, tp):
            pltpu.sync_copy(tf.at[pl.ds(s * ROWS_T, ROWS_T), :],
                            acc.at[pl.ds(s * ROWS_T, ROWS_T), :])
            plsc.subcore_barrier()
            if class_split:
                tile_row0 = s * rows_per_tile
            else:
                tile_row0 = (c * NT + s) * rows_per_tile

            def idesc(g, p):
                return pltpu.make_async_copy(
                    idx_hbm.at[pl.ds(tile_row0 + g * GROUP, GROUP), :, :],
                    ibufs[p], isems[p])

            def gfire(ib, j, p):
                return pltpu.async_copy(tp.at[ib.at[j, 0]], pbufs[p],
                                        gsems[p])

            idesc(0, 0).start()

            def outer(k2, carry):
                for p in range(2):
                    g = 2 * k2 + p
                    idesc(g, p).wait()
                    gnxt = jnp.minimum(g + 1, ngroups - 1)
                    idesc(gnxt, 1 - p).start()
                    ib = ibufs[p]
                    gd = {0: gfire(ib, 0, 0), 1: gfire(ib, 1, 1)}
                    sd = {}
                    for j in range(GROUP):
                        b = j % 2
                        gd[j].wait()
                        if j >= 1:
                            sd[j - 1].wait()
                        cvt(pbufs[b])
                        if j + 2 < GROUP:
                            gd[j + 2] = gfire(ib, j + 2, b)
                        sd[j] = pltpu.async_copy(fbuf, acc.at[ib.at[j, 1]],
                                                 ssem, add=True)
                    sd[GROUP - 1].wait()
                return carry

            lax.fori_loop(0, ngroups // 2, outer, 0)
            idesc(ngroups - 1, 0).wait()
            plsc.subcore_barrier()
            pltpu.sync_copy(acc.at[pl.ds(s * ROWS_T, ROWS_T), :],
                            out_hbm.at[c, pl.ds(s * ROWS_T, ROWS_T), :])

        if class_split:
            @pl.when(c == 0)
            def _():
                run(tf_a, tp_a)

            @pl.when(c != 0)
            def _():
                run(tf_b, tp_b)
        else:
            run(tf_a, tp_a)

    return functools.partial(
        pl.kernel,
        out_type=jax.ShapeDtypeStruct((NSC, NP, H), jnp.float32),
        mesh=_mesh(),
        compiler_params=pltpu.CompilerParams(use_tc_tiling_on_sc=False,
                                             needs_layout_passes=False),
        scratch_types=[
            pltpu.VMEM((GROUP, 2, 128), jnp.int32),
            pltpu.VMEM((GROUP, 2, 128), jnp.int32),
            pltpu.VMEM((K, H // 2), jnp.int32),
            pltpu.VMEM((K, H // 2), jnp.int32),
            pltpu.VMEM((K, H), jnp.float32),
            pltpu.SemaphoreType.DMA,
            pltpu.SemaphoreType.DMA,
            pltpu.SemaphoreType.DMA,
            pltpu.SemaphoreType.DMA,
            pltpu.SemaphoreType.DMA,
            pltpu.VMEM_SHARED((NP, H), jnp.float32),
        ],
    )(body)


def _pack_tab(sf):
    sb = sf.astype(jnp.bfloat16)
    pk = jnp.stack([sb[..., :64], sb[..., 64:]], axis=-1)
    return jax.lax.bitcast_convert_type(pk, jnp.int32)


def _pack_tab16(sf):
    q = jnp.clip(jnp.round(sf * 65535.0), 0.0, 65535.0).astype(jnp.int32)
    return q[..., :64] | (q[..., 64:] << 16)


_apass_shared_pk = _make_apass_pk(class_split=False)
_apass_class_pk16 = _make_apass_pk(class_split=True, unpack_kind='u16')

_apass_class = _make_apass(class_split=True)


def _prescale_body(x_ref, d0_ref, d1_ref, dinv_ref, s0_ref):
    deg = d0_ref[...] + d1_ref[...] + 1.0
    dinv = 1.0 / jnp.sqrt(deg)
    dinv_ref[...] = dinv
    s0_ref[...] = x_ref[...] * dinv


def _prescale(xp, d0, d1):
    return pl.pallas_call(
        _prescale_body,
        grid=(NB,),
        in_specs=[
            pl.BlockSpec((RB, H), lambda i: (i, 0)),
            pl.BlockSpec((RB, 1), lambda i: (i, 0)),
            pl.BlockSpec((RB, 1), lambda i: (i, 0)),
        ],
        out_specs=[
            pl.BlockSpec((RB, 1), lambda i: (i, 0)),
            pl.BlockSpec((RB, H), lambda i: (i, 0)),
        ],
        out_shape=[
            jax.ShapeDtypeStruct((NP, 1), jnp.float32),
            jax.ShapeDtypeStruct((NP, H), jnp.float32),
        ],
    )(xp, d0, d1)


def _l2relu(q):
    r2 = jnp.sum(q * q, axis=1, keepdims=True)
    nrm = jnp.maximum(jnp.sqrt(r2), 1e-12)
    return jnp.maximum(q / nrm, 0.0)


def _dense1_body(ua_ref, ub_ref, s0_ref, dinv_ref, w_ref, b_ref, out_ref):
    dinv = dinv_ref[...]
    t = (ua_ref[0] + ub_ref[0] - s0_ref[...]) * dinv
    q = jnp.dot(t, w_ref[0], preferred_element_type=jnp.float32) + b_ref[0]
    out_ref[0] = _l2relu(q) * dinv


def _dense1(u1p, s0, dinv, w, b):
    return pl.pallas_call(
        _dense1_body,
        grid=(C, NB),
        in_specs=[
            pl.BlockSpec((1, RB, H), lambda c, i: (0, i, 0)),
            pl.BlockSpec((1, RB, H), lambda c, i: (1, i, 0)),
            pl.BlockSpec((RB, H), lambda c, i: (i, 0)),
            pl.BlockSpec((RB, 1), lambda c, i: (i, 0)),
            pl.BlockSpec((1, H, H), lambda c, i: (c, 0, 0)),
            pl.BlockSpec((1, 1, H), lambda c, i: (c, 0, 0)),
        ],
        out_specs=pl.BlockSpec((1, RB, H), lambda c, i: (c, i, 0)),
        out_shape=jax.ShapeDtypeStruct((C, NP, H), jnp.float32),
    )(u1p, u1p, s0, dinv, w, b)


def _make_dense23(prescale_out):
    def body(u_ref, dinv_ref, w_ref, b_ref, out_ref):
        dinv = dinv_ref[...]
        t = u_ref[0] * dinv
        q = jnp.dot(t, w_ref[0], preferred_element_type=jnp.float32) + b_ref[0]
        h = _l2relu(q)
        out_ref[0] = h * dinv if prescale_out else h

    def call(u, dinv, w, b):
        return pl.pallas_call(
            body,
            grid=(C, NB),
            in_specs=[
                pl.BlockSpec((1, RB, H), lambda c, i: (c, i, 0)),
                pl.BlockSpec((RB, 1), lambda c, i: (i, 0)),
                pl.BlockSpec((1, H, H), lambda c, i: (c, 0, 0)),
                pl.BlockSpec((1, 1, H), lambda c, i: (c, 0, 0)),
            ],
            out_specs=pl.BlockSpec((1, RB, H), lambda c, i: (c, i, 0)),
            out_shape=jax.ShapeDtypeStruct((C, NP, H), jnp.float32),
        )(u, dinv, w, b)

    return call


_dense2 = _make_dense23(prescale_out=True)
_dense3 = _make_dense23(prescale_out=False)


def _pool_body(h_ref, batch_ref, w1_ref, b1_ref, w2_ref, b2_ref, out_ref):
    hb = h_ref[0]
    bt = batch_ref[0]
    gids = lax.broadcasted_iota(jnp.int32, (G, NP), 0)
    oh = (bt == gids).astype(jnp.float32)
    pooled = jnp.dot(oh, hb, preferred_element_type=jnp.float32)
    z = jnp.dot(pooled, w1_ref[0], preferred_element_type=jnp.float32)
    z = jnp.maximum(z + b1_ref[0], 0.0)
    o = jnp.sum(z * w2_ref[0], axis=1) + b2_ref[0, 0]
    out_ref[0, 0] = o


def _pool_head(h3, batchp, w1, b1, w2t, b2):
    return pl.pallas_call(
        _pool_body,
        grid=(C,),
        in_specs=[
            pl.BlockSpec((1, NP, H), lambda c: (c, 0, 0)),
            pl.BlockSpec((1, 1, NP), lambda c: (0, 0, 0)),
            pl.BlockSpec((1, H, H), lambda c: (c, 0, 0)),
            pl.BlockSpec((1, 1, H), lambda c: (c, 0, 0)),
            pl.BlockSpec((1, 1, H), lambda c: (c, 0, 0)),
            pl.BlockSpec((1, 1, 1), lambda c: (c, 0, 0)),
        ],
        out_specs=pl.BlockSpec((1, 1, G), lambda c: (c, 0, 0)),
        out_shape=jax.ShapeDtypeStruct((C, 1, G), jnp.float32),
    )(h3, batchp, w1, b1, w2t, b2)


def kernel(x, edge_index, batch, conv_W0, conv_b0, conv_W1, conv_b1,
           conv_W2, conv_b2, lin1_W, lin1_b, lin2_W, lin2_b):
    pad_e = EP - E
    srcp = jnp.concatenate(
        [edge_index[0], jnp.zeros((pad_e,), edge_index.dtype)]).reshape(ER, 128)
    dstp = jnp.concatenate(
        [edge_index[1], jnp.full((pad_e,), N, edge_index.dtype)]).reshape(ER, 128)
    idx2 = jnp.stack([srcp, dstp], axis=1)
    xp = jnp.pad(x, ((0, NP - N), (0, 0)))
    batchp = jnp.pad(batch, (0, NP - N), constant_values=G).reshape(1, 1, NP)

    degp = _deg_pass(idx2)
    d0 = degp[0].reshape(NP, 1)
    d1 = degp[1].reshape(NP, 1)
    dinv, s0 = _prescale(xp, d0, d1)

    b0 = conv_b0.reshape(C, 1, H)
    b1 = conv_b1.reshape(C, 1, H)
    b2 = conv_b2.reshape(C, 1, H)
    l1b = lin1_b.reshape(C, 1, H)
    w2t = jnp.transpose(lin2_W, (0, 2, 1))
    l2b = lin2_b.reshape(C, 1, 1)

    u1p = _apass_shared_pk(s0, _pack_tab(s0), idx2)
    s1 = _dense1(u1p, s0, dinv, conv_W0, b0)
    s1p = _pack_tab16(s1)
    u2 = _apass_class_pk16(s1[0], s1[1], s1p[0], s1p[1], idx2)
    s2 = _dense2(u2, dinv, conv_W1, b1)
    s2p = _pack_tab16(s2)
    u3 = _apass_class_pk16(s2[0], s2[1], s2p[0], s2p[1], idx2)
    h3 = _dense3(u3, dinv, conv_W2, b2)
    out = _pool_head(h3, batchp, lin1_W, l1b, w2t, l2b)
    return jnp.transpose(out[:, 0, :], (1, 0))


# speedup vs baseline: 1.1810x; 1.0058x over previous
import functools

import jax
import jax.numpy as jnp
from jax import lax
from jax.experimental import pallas as pl
from jax.experimental.pallas import tpu as pltpu
from jax.experimental.pallas import tpu_sc as plsc

N = 10000
E = 320000
H = 128
C = 2
G = 64

NT = 16
NSC = 2
NP = 10240
ROWS_T = NP // NT
EP = 327680
ER = EP // 128
K = 128
GROUP = 10
KR = 2
RB = 1024
NB = NP // RB


def _mesh():
    return plsc.VectorSubcoreMesh(core_axis_name="c", subcore_axis_name="s")


def _deg_body(idx_hbm, out_hbm, idx_v, ones_v, zb_v, acc):
    c = lax.axis_index("c")
    s = lax.axis_index("s")
    w = c * NT + s
    for j in range(128 // 16):
        ones_v[pl.ds(j * 16, 16)] = jnp.full((16,), 1.0, jnp.float32)

    def zf(i, carry):
        zb_v[pl.ds(i * 16, 16)] = jnp.zeros((16,), jnp.float32)
        return carry

    lax.fori_loop(0, ROWS_T // 16, zf, 0)
    pltpu.sync_copy(zb_v, acc.at[pl.ds(s * ROWS_T, ROWS_T)])
    plsc.subcore_barrier()

    rows_per_tile = ER // (NSC * NT)

    def chunk(i, carry):
        r0 = w * rows_per_tile + i * KR
        pltpu.sync_copy(idx_hbm.at[pl.ds(r0, KR), 1, :], idx_v)
        for j in range(KR):
            pltpu.sync_copy(ones_v, acc.at[idx_v.at[j]], add=True)
        return carry

    lax.fori_loop(0, rows_per_tile // KR, chunk, 0)
    plsc.subcore_barrier()
    pltpu.sync_copy(acc.at[pl.ds(s * ROWS_T, ROWS_T)],
                    out_hbm.at[c, pl.ds(s * ROWS_T, ROWS_T)])


_deg_pass = functools.partial(
    pl.kernel,
    out_type=jax.ShapeDtypeStruct((NSC, NP), jnp.float32),
    mesh=_mesh(),
    scratch_types=[
        pltpu.VMEM((KR, 128), jnp.int32),
        pltpu.VMEM((128,), jnp.float32),
        pltpu.VMEM((ROWS_T,), jnp.float32),
        pltpu.VMEM_SHARED((NP,), jnp.float32),
    ],
)(_deg_body)


def _make_apass(class_split):
    rows_per_tile = ER // NT if class_split else ER // (NSC * NT)
    ngroups = rows_per_tile // GROUP

    def body(*refs):
        if class_split:
            tab_a, tab_b, idx_hbm, out_hbm = refs[:4]
            scratch = refs[4:]
        else:
            tab_a, idx_hbm, out_hbm = refs[:3]
            scratch = refs[3:]
            tab_b = tab_a
        (ib0, ib1, rows0, rows1, isem0, isem1,
         gsem00, gsem01, gsem10, gsem11, ssem0, ssem1, acc) = scratch
        ibufs, isems = (ib0, ib1), (isem0, isem1)
        rbufs = (rows0, rows1)
        gsems = ((gsem00, gsem01), (gsem10, gsem11))
        ssems = (ssem0, ssem1)
        c = lax.axis_index("c")
        s = lax.axis_index("s")

        def run(tab):
            pltpu.sync_copy(tab.at[pl.ds(s * ROWS_T, ROWS_T), :],
                            acc.at[pl.ds(s * ROWS_T, ROWS_T), :])
            plsc.subcore_barrier()
            if class_split:
                tile_row0 = s * rows_per_tile
            else:
                tile_row0 = (c * NT + s) * rows_per_tile

            def idesc(g, p):
                return pltpu.make_async_copy(
                    idx_hbm.at[pl.ds(tile_row0 + g * GROUP, GROUP), :, :],
                    ibufs[p], isems[p])

            def gfire(ib, j, p):
                return (
                    pltpu.async_copy(tab.at[ib.at[j, 0, pl.ds(0, 64)]],
                                     rbufs[p].at[pl.ds(0, 64), :],
                                     gsems[p][0]),
                    pltpu.async_copy(tab.at[ib.at[j, 0, pl.ds(64, 64)]],
                                     rbufs[p].at[pl.ds(64, 64), :],
                                     gsems[p][1]),
                )

            idesc(0, 0).start()

            def outer(k2, carry):
                for p in range(2):
                    g = 2 * k2 + p
                    idesc(g, p).wait()
                    gnxt = jnp.minimum(g + 1, ngroups - 1)
                    idesc(gnxt, 1 - p).start()
                    ib = ibufs[p]
                    gd = {0: gfire(ib, 0, 0), 1: gfire(ib, 1, 1)}
                    sd = {}
                    for j in range(GROUP):
                        b = j % 2
                        for d in gd[j]:
                            d.wait()
                        sd[j] = pltpu.async_copy(rbufs[b],
                                                 acc.at[ib.at[j, 1]],
                                                 ssems[b], add=True)
                        if j + 2 < GROUP:
                            sd[j].wait()
                            gd[j + 2] = gfire(ib, j + 2, b)
                    sd[GROUP - 2].wait()
                    sd[GROUP - 1].wait()
                return carry

            lax.fori_loop(0, ngroups // 2, outer, 0)
            idesc(ngroups - 1, 0).wait()
            plsc.subcore_barrier()
            pltpu.sync_copy(acc.at[pl.ds(s * ROWS_T, ROWS_T), :],
                            out_hbm.at[c, pl.ds(s * ROWS_T, ROWS_T), :])

        if class_split:
            @pl.when(c == 0)
            def _():
                run(tab_a)

            @pl.when(c != 0)
            def _():
                run(tab_b)
        else:
            run(tab_a)

    return functools.partial(
        pl.kernel,
        out_type=jax.ShapeDtypeStruct((NSC, NP, H), jnp.float32),
        mesh=_mesh(),
        scratch_types=[
            pltpu.VMEM((GROUP, 2, 128), jnp.int32),
            pltpu.VMEM((GROUP, 2, 128), jnp.int32),
            pltpu.VMEM((K, H), jnp.float32),
            pltpu.VMEM((K, H), jnp.float32),
            pltpu.SemaphoreType.DMA,
            pltpu.SemaphoreType.DMA,
            pltpu.SemaphoreType.DMA,
            pltpu.SemaphoreType.DMA,
            pltpu.SemaphoreType.DMA,
            pltpu.SemaphoreType.DMA,
            pltpu.SemaphoreType.DMA,
            pltpu.SemaphoreType.DMA,
            pltpu.VMEM_SHARED((NP, H), jnp.float32),
        ],
    )(body)


def _make_apass_pk(class_split, unpack_kind='bf16'):
    rows_per_tile = ER // NT if class_split else ER // (NSC * NT)
    ngroups = rows_per_tile // GROUP

    def body(*refs):
        if class_split:
            tf_a, tf_b, tp_a, tp_b, idx_hbm, out_hbm = refs[:6]
            scratch = refs[6:]
        else:
            tf_a, tp_a, idx_hbm, out_hbm = refs[:4]
            scratch = refs[4:]
            tf_b, tp_b = tf_a, tp_a
        (ib0, ib1, pk0, pk1, fbuf, isem0, isem1,
         gsem0, gsem1, ssem, acc) = scratch
        ibufs, isems = (ib0, ib1), (isem0, isem1)
        pbufs, gsems = (pk0, pk1), (gsem0, gsem1)
        c = lax.axis_index("c")
        s = lax.axis_index("s")

        def cvt(pb):
            if unpack_kind == 'bf16':
                @plsc.parallel_loop(0, K, unroll=4)
                def _(r):
                    for k in range(4):
                        w = plsc.bitcast(pb[r, pl.ds(16 * k, 16)],
                                         jnp.bfloat16)
                        lo, hi = plsc.unpack(w, format=plsc.PackFormat.INTERLEAVED)
                        fbuf[r, pl.ds(16 * k, 16)] = lo
                        fbuf[r, pl.ds(64 + 16 * k, 16)] = hi
            else:
                if unpack_kind == 'u16':
                    inv, off = jnp.float32(1.0 / 65535.0), jnp.float32(0.0)
                else:
                    inv, off = jnp.float32(1.0 / 4096.0), jnp.float32(8.0)

                @plsc.parallel_loop(0, K, unroll=4)
                def _(r):
                    for k in range(4):
                        w = pb[r, pl.ds(16 * k, 16)]
                        lo = (w & 0xFFFF).astype(jnp.float32) * inv - off
                        hi = ((w >> 16) & 0xFFFF).astype(jnp.float32) * inv - off
                        fbuf[r, pl.ds(16 * k, 16)] = lo
                        fbuf[r, pl.ds(64 + 16 * k, 16)] = hi

        def run(tf, tp):
            pltpu.sync_copy(tf.at[pl.ds(s * ROWS_T, ROWS_T), :],
                            acc.at[pl.ds(s * ROWS_T, ROWS_T), :])
            plsc.subcore_barrier()
            if class_split:
                tile_row0 = s * rows_per_tile
            else:
                tile_row0 = (c * NT + s) * rows_per_tile

            def idesc(g, p):
                return pltpu.make_async_copy(
                    idx_hbm.at[pl.ds(tile_row0 + g * GROUP, GROUP), :, :],
                    ibufs[p], isems[p])

            def gfire(ib, j, p):
                return pltpu.async_copy(tp.at[ib.at[j, 0]], pbufs[p],
                                        gsems[p])

            idesc(0, 0).start()

            def outer(k2, carry):
                for p in range(2):
                    g = 2 * k2 + p
                    idesc(g, p).wait()
                    gnxt = jnp.minimum(g + 1, ngroups - 1)
                    idesc(gnxt, 1 - p).start()
                    ib = ibufs[p]
                    gd = {0: gfire(ib, 0, 0), 1: gfire(ib, 1, 1)}
                    sd = {}
                    for j in range(GROUP):
                        b = j % 2
                        gd[j].wait()
                        if j >= 1:
                            sd[j - 1].wait()
                        cvt(pbufs[b])
                        if j + 2 < GROUP:
                            gd[j + 2] = gfire(ib, j + 2, b)
                        sd[j] = pltpu.async_copy(fbuf, acc.at[ib.at[j, 1]],
                                                 ssem, add=True)
                    sd[GROUP - 1].wait()
                return carry

            lax.fori_loop(0, ngroups // 2, outer, 0)
            idesc(ngroups - 1, 0).wait()
            plsc.subcore_barrier()
            pltpu.sync_copy(acc.at[pl.ds(s * ROWS_T, ROWS_T), :],
                            out_hbm.at[c, pl.ds(s * ROWS_T, ROWS_T), :])

        if class_split:
            @pl.when(c == 0)
            def _():
                run(tf_a, tp_a)

            @pl.when(c != 0)
            def _():
                run(tf_b, tp_b)
        else:
            run(tf_a, tp_a)

    return functools.partial(
        pl.kernel,
        out_type=jax.ShapeDtypeStruct((NSC, NP, H), jnp.float32),
        mesh=_mesh(),
        compiler_params=pltpu.CompilerParams(use_tc_tiling_on_sc=False,
                                             needs_layout_passes=False),
        scratch_types=[
            pltpu.VMEM((GROUP, 2, 128), jnp.int32),
            pltpu.VMEM((GROUP, 2, 128), jnp.int32),
            pltpu.VMEM((K, H // 2), jnp.int32),
            pltpu.VMEM((K, H // 2), jnp.int32),
            pltpu.VMEM((K, H), jnp.float32),
            pltpu.SemaphoreType.DMA,
            pltpu.SemaphoreType.DMA,
            pltpu.SemaphoreType.DMA,
            pltpu.SemaphoreType.DMA,
            pltpu.SemaphoreType.DMA,
            pltpu.VMEM_SHARED((NP, H), jnp.float32),
        ],
    )(body)


def _pack_tab16(sf):
    q = jnp.clip(jnp.round(sf * 65535.0), 0.0, 65535.0).astype(jnp.int32)
    return q[..., :64] | (q[..., 64:] << 16)


def _pack_tab16c(sf):
    q = jnp.clip(jnp.round((sf + 8.0) * 4096.0), 0.0, 65535.0).astype(jnp.int32)
    return q[..., :64] | (q[..., 64:] << 16)


_apass_shared_pk16c = _make_apass_pk(class_split=False, unpack_kind='u16c')
_apass_class_pk16 = _make_apass_pk(class_split=True, unpack_kind='u16')

_apass_class = _make_apass(class_split=True)


def _prescale_body(x_ref, d0_ref, d1_ref, dinv_ref, s0_ref):
    deg = d0_ref[...] + d1_ref[...] + 1.0
    dinv = 1.0 / jnp.sqrt(deg)
    dinv_ref[...] = dinv
    s0_ref[...] = x_ref[...] * dinv


def _prescale(xp, d0, d1):
    return pl.pallas_call(
        _prescale_body,
        grid=(NB,),
        in_specs=[
            pl.BlockSpec((RB, H), lambda i: (i, 0)),
            pl.BlockSpec((RB, 1), lambda i: (i, 0)),
            pl.BlockSpec((RB, 1), lambda i: (i, 0)),
        ],
        out_specs=[
            pl.BlockSpec((RB, 1), lambda i: (i, 0)),
            pl.BlockSpec((RB, H), lambda i: (i, 0)),
        ],
        out_shape=[
            jax.ShapeDtypeStruct((NP, 1), jnp.float32),
            jax.ShapeDtypeStruct((NP, H), jnp.float32),
        ],
    )(xp, d0, d1)


def _l2relu(q):
    r2 = jnp.sum(q * q, axis=1, keepdims=True)
    nrm = jnp.maximum(jnp.sqrt(r2), 1e-12)
    return jnp.maximum(q / nrm, 0.0)


def _dense1_body(ua_ref, ub_ref, s0_ref, dinv_ref, w_ref, b_ref, out_ref):
    dinv = dinv_ref[...]
    t = (ua_ref[0] + ub_ref[0] - s0_ref[...]) * dinv
    q = jnp.dot(t, w_ref[0], preferred_element_type=jnp.float32) + b_ref[0]
    out_ref[0] = _l2relu(q) * dinv


def _dense1(u1p, s0, dinv, w, b):
    return pl.pallas_call(
        _dense1_body,
        grid=(C, NB),
        in_specs=[
            pl.BlockSpec((1, RB, H), lambda c, i: (0, i, 0)),
            pl.BlockSpec((1, RB, H), lambda c, i: (1, i, 0)),
            pl.BlockSpec((RB, H), lambda c, i: (i, 0)),
            pl.BlockSpec((RB, 1), lambda c, i: (i, 0)),
            pl.BlockSpec((1, H, H), lambda c, i: (c, 0, 0)),
            pl.BlockSpec((1, 1, H), lambda c, i: (c, 0, 0)),
        ],
        out_specs=pl.BlockSpec((1, RB, H), lambda c, i: (c, i, 0)),
        out_shape=jax.ShapeDtypeStruct((C, NP, H), jnp.float32),
    )(u1p, u1p, s0, dinv, w, b)


def _make_dense23(prescale_out):
    def body(u_ref, dinv_ref, w_ref, b_ref, out_ref):
        dinv = dinv_ref[...]
        t = u_ref[0] * dinv
        q = jnp.dot(t, w_ref[0], preferred_element_type=jnp.float32) + b_ref[0]
        h = _l2relu(q)
        out_ref[0] = h * dinv if prescale_out else h

    def call(u, dinv, w, b):
        return pl.pallas_call(
            body,
            grid=(C, NB),
            in_specs=[
                pl.BlockSpec((1, RB, H), lambda c, i: (c, i, 0)),
                pl.BlockSpec((RB, 1), lambda c, i: (i, 0)),
                pl.BlockSpec((1, H, H), lambda c, i: (c, 0, 0)),
                pl.BlockSpec((1, 1, H), lambda c, i: (c, 0, 0)),
            ],
            out_specs=pl.BlockSpec((1, RB, H), lambda c, i: (c, i, 0)),
            out_shape=jax.ShapeDtypeStruct((C, NP, H), jnp.float32),
        )(u, dinv, w, b)

    return call


_dense2 = _make_dense23(prescale_out=True)
_dense3 = _make_dense23(prescale_out=False)


def _pool_body(h_ref, batch_ref, w1_ref, b1_ref, w2_ref, b2_ref, out_ref):
    hb = h_ref[0]
    bt = batch_ref[0]
    gids = lax.broadcasted_iota(jnp.int32, (G, NP), 0)
    oh = (bt == gids).astype(jnp.float32)
    pooled = jnp.dot(oh, hb, preferred_element_type=jnp.float32)
    z = jnp.dot(pooled, w1_ref[0], preferred_element_type=jnp.float32)
    z = jnp.maximum(z + b1_ref[0], 0.0)
    o = jnp.sum(z * w2_ref[0], axis=1) + b2_ref[0, 0]
    out_ref[0, 0] = o


def _pool_head(h3, batchp, w1, b1, w2t, b2):
    return pl.pallas_call(
        _pool_body,
        grid=(C,),
        in_specs=[
            pl.BlockSpec((1, NP, H), lambda c: (c, 0, 0)),
            pl.BlockSpec((1, 1, NP), lambda c: (0, 0, 0)),
            pl.BlockSpec((1, H, H), lambda c: (c, 0, 0)),
            pl.BlockSpec((1, 1, H), lambda c: (c, 0, 0)),
            pl.BlockSpec((1, 1, H), lambda c: (c, 0, 0)),
            pl.BlockSpec((1, 1, 1), lambda c: (c, 0, 0)),
        ],
        out_specs=pl.BlockSpec((1, 1, G), lambda c: (c, 0, 0)),
        out_shape=jax.ShapeDtypeStruct((C, 1, G), jnp.float32),
    )(h3, batchp, w1, b1, w2t, b2)


def kernel(x, edge_index, batch, conv_W0, conv_b0, conv_W1, conv_b1,
           conv_W2, conv_b2, lin1_W, lin1_b, lin2_W, lin2_b):
    pad_e = EP - E
    srcp = jnp.concatenate(
        [edge_index[0], jnp.zeros((pad_e,), edge_index.dtype)]).reshape(ER, 128)
    dstp = jnp.concatenate(
        [edge_index[1], jnp.full((pad_e,), N, edge_index.dtype)]).reshape(ER, 128)
    idx2 = jnp.stack([srcp, dstp], axis=1)
    xp = jnp.pad(x, ((0, NP - N), (0, 0)))
    batchp = jnp.pad(batch, (0, NP - N), constant_values=G).reshape(1, 1, NP)

    degp = _deg_pass(idx2)
    d0 = degp[0].reshape(NP, 1)
    d1 = degp[1].reshape(NP, 1)
    dinv, s0 = _prescale(xp, d0, d1)

    b0 = conv_b0.reshape(C, 1, H)
    b1 = conv_b1.reshape(C, 1, H)
    b2 = conv_b2.reshape(C, 1, H)
    l1b = lin1_b.reshape(C, 1, H)
    w2t = jnp.transpose(lin2_W, (0, 2, 1))
    l2b = lin2_b.reshape(C, 1, 1)

    u1p = _apass_shared_pk16c(s0, _pack_tab16c(s0), idx2)
    s1 = _dense1(u1p, s0, dinv, conv_W0, b0)
    s1p = _pack_tab16(s1)
    u2 = _apass_class_pk16(s1[0], s1[1], s1p[0], s1p[1], idx2)
    s2 = _dense2(u2, dinv, conv_W1, b1)
    s2p = _pack_tab16(s2)
    u3 = _apass_class_pk16(s2[0], s2[1], s2p[0], s2p[1], idx2)
    h3 = _dense3(u3, dinv, conv_W2, b2)
    out = _pool_head(h3, batchp, lin1_W, l1b, w2t, l2b)
    return jnp.transpose(out[:, 0, :], (1, 0))


# speedup vs baseline: 1.1813x; 1.0003x over previous
import functools

import jax
import jax.numpy as jnp
from jax import lax
from jax.experimental import pallas as pl
from jax.experimental.pallas import tpu as pltpu
from jax.experimental.pallas import tpu_sc as plsc

N = 10000
E = 320000
H = 128
C = 2
G = 64

NT = 16
NSC = 2
NP = 10240
ROWS_T = NP // NT
EP = 327680
ER = EP // 128
K = 128
GROUP = 10
KR = 2
RB = 1024
NB = NP // RB


def _mesh():
    return plsc.VectorSubcoreMesh(core_axis_name="c", subcore_axis_name="s")


def _deg_body(idx_hbm, out_hbm, idx_v, ones_v, zb_v, acc):
    c = lax.axis_index("c")
    s = lax.axis_index("s")
    w = c * NT + s
    for j in range(128 // 16):
        ones_v[pl.ds(j * 16, 16)] = jnp.full((16,), 1.0, jnp.float32)

    def zf(i, carry):
        zb_v[pl.ds(i * 16, 16)] = jnp.zeros((16,), jnp.float32)
        return carry

    lax.fori_loop(0, ROWS_T // 16, zf, 0)
    pltpu.sync_copy(zb_v, acc.at[pl.ds(s * ROWS_T, ROWS_T)])
    plsc.subcore_barrier()

    rows_per_tile = ER // (NSC * NT)

    def chunk(i, carry):
        r0 = w * rows_per_tile + i * KR
        pltpu.sync_copy(idx_hbm.at[pl.ds(r0, KR), 1, :], idx_v)
        for j in range(KR):
            pltpu.sync_copy(ones_v, acc.at[idx_v.at[j]], add=True)
        return carry

    lax.fori_loop(0, rows_per_tile // KR, chunk, 0)
    plsc.subcore_barrier()
    pltpu.sync_copy(acc.at[pl.ds(s * ROWS_T, ROWS_T)],
                    out_hbm.at[c, pl.ds(s * ROWS_T, ROWS_T)])


_deg_pass = functools.partial(
    pl.kernel,
    out_type=jax.ShapeDtypeStruct((NSC, NP), jnp.float32),
    mesh=_mesh(),
    scratch_types=[
        pltpu.VMEM((KR, 128), jnp.int32),
        pltpu.VMEM((128,), jnp.float32),
        pltpu.VMEM((ROWS_T,), jnp.float32),
        pltpu.VMEM_SHARED((NP,), jnp.float32),
    ],
)(_deg_body)


def _make_apass(class_split):
    rows_per_tile = ER // NT if class_split else ER // (NSC * NT)
    ngroups = rows_per_tile // GROUP

    def body(*refs):
        if class_split:
            tab_a, tab_b, idx_hbm, out_hbm = refs[:4]
            scratch = refs[4:]
        else:
            tab_a, idx_hbm, out_hbm = refs[:3]
            scratch = refs[3:]
            tab_b = tab_a
        (ib0, ib1, rows0, rows1, isem0, isem1,
         gsem00, gsem01, gsem10, gsem11, ssem0, ssem1, acc) = scratch
        ibufs, isems = (ib0, ib1), (isem0, isem1)
        rbufs = (rows0, rows1)
        gsems = ((gsem00, gsem01), (gsem10, gsem11))
        ssems = (ssem0, ssem1)
        c = lax.axis_index("c")
        s = lax.axis_index("s")

        def run(tab):
            pltpu.sync_copy(tab.at[pl.ds(s * ROWS_T, ROWS_T), :],
                            acc.at[pl.ds(s * ROWS_T, ROWS_T), :])
            plsc.subcore_barrier()
            if class_split:
                tile_row0 = s * rows_per_tile
            else:
                tile_row0 = (c * NT + s) * rows_per_tile

            def idesc(g, p):
                return pltpu.make_async_copy(
                    idx_hbm.at[pl.ds(tile_row0 + g * GROUP, GROUP), :, :],
                    ibufs[p], isems[p])

            def gfire(ib, j, p):
                return (
                    pltpu.async_copy(tab.at[ib.at[j, 0, pl.ds(0, 64)]],
                                     rbufs[p].at[pl.ds(0, 64), :],
                                     gsems[p][0]),
                    pltpu.async_copy(tab.at[ib.at[j, 0, pl.ds(64, 64)]],
                                     rbufs[p].at[pl.ds(64, 64), :],
                                     gsems[p][1]),
                )

            idesc(0, 0).start()

            def outer(k2, carry):
                for p in range(2):
                    g = 2 * k2 + p
                    idesc(g, p).wait()
                    gnxt = jnp.minimum(g + 1, ngroups - 1)
                    idesc(gnxt, 1 - p).start()
                    ib = ibufs[p]
                    gd = {0: gfire(ib, 0, 0), 1: gfire(ib, 1, 1)}
                    sd = {}
                    for j in range(GROUP):
                        b = j % 2
                        for d in gd[j]:
                            d.wait()
                        sd[j] = pltpu.async_copy(rbufs[b],
                                                 acc.at[ib.at[j, 1]],
                                                 ssems[b], add=True)
                        if j + 2 < GROUP:
                            sd[j].wait()
                            gd[j + 2] = gfire(ib, j + 2, b)
                    sd[GROUP - 2].wait()
                    sd[GROUP - 1].wait()
                return carry

            lax.fori_loop(0, ngroups // 2, outer, 0)
            idesc(ngroups - 1, 0).wait()
            plsc.subcore_barrier()
            pltpu.sync_copy(acc.at[pl.ds(s * ROWS_T, ROWS_T), :],
                            out_hbm.at[c, pl.ds(s * ROWS_T, ROWS_T), :])

        if class_split:
            @pl.when(c == 0)
            def _():
                run(tab_a)

            @pl.when(c != 0)
            def _():
                run(tab_b)
        else:
            run(tab_a)

    return functools.partial(
        pl.kernel,
        out_type=jax.ShapeDtypeStruct((NSC, NP, H), jnp.float32),
        mesh=_mesh(),
        scratch_types=[
            pltpu.VMEM((GROUP, 2, 128), jnp.int32),
            pltpu.VMEM((GROUP, 2, 128), jnp.int32),
            pltpu.VMEM((K, H), jnp.float32),
            pltpu.VMEM((K, H), jnp.float32),
            pltpu.SemaphoreType.DMA,
            pltpu.SemaphoreType.DMA,
            pltpu.SemaphoreType.DMA,
            pltpu.SemaphoreType.DMA,
            pltpu.SemaphoreType.DMA,
            pltpu.SemaphoreType.DMA,
            pltpu.SemaphoreType.DMA,
            pltpu.SemaphoreType.DMA,
            pltpu.VMEM_SHARED((NP, H), jnp.float32),
        ],
    )(body)


def _make_apass_pk(class_split, unpack_kind='u16'):
    rows_per_tile = ER // NT if class_split else ER // (NSC * NT)
    ngroups = rows_per_tile // GROUP

    def body(*refs):
        if class_split:
            tf_a, tf_b, tp_a, tp_b, idx_hbm, out_hbm = refs[:6]
            scratch = refs[6:]
        else:
            tf_a, tp_a, idx_hbm, out_hbm = refs[:4]
            scratch = refs[4:]
            tf_b, tp_b = tf_a, tp_a
        (ib0, ib1, pk0, pk1, fbuf, isem0, isem1,
         gsem0, gsem1, ssem, acc) = scratch
        ibufs, isems = (ib0, ib1), (isem0, isem1)
        pbufs, gsems = (pk0, pk1), (gsem0, gsem1)
        c = lax.axis_index("c")
        s = lax.axis_index("s")

        def cvt(pb):
            if unpack_kind == 'u16':
                inv, off = jnp.float32(1.0 / 65535.0), jnp.float32(0.0)
            else:
                inv, off = jnp.float32(1.0 / 4096.0), jnp.float32(8.0)

            @plsc.parallel_loop(0, K, unroll=4)
            def _(r):
                for k in range(4):
                    w = pb[r, pl.ds(16 * k, 16)]
                    lo = (w & 0xFFFF).astype(jnp.float32) * inv - off
                    hi = ((w >> 16) & 0xFFFF).astype(jnp.float32) * inv - off
                    fbuf[r, pl.ds(16 * k, 16)] = lo
                    fbuf[r, pl.ds(64 + 16 * k, 16)] = hi

        def run(tf, tp):
            pltpu.sync_copy(tf.at[pl.ds(s * ROWS_T, ROWS_T), :],
                            acc.at[pl.ds(s * ROWS_T, ROWS_T), :])
            plsc.subcore_barrier()
            if class_split:
                tile_row0 = s * rows_per_tile
            else:
                tile_row0 = (c * NT + s) * rows_per_tile

            def idesc(g, p):
                return pltpu.make_async_copy(
                    idx_hbm.at[pl.ds(tile_row0 + g * GROUP, GROUP), :, :],
                    ibufs[p], isems[p])

            def gfire(ib, j, p):
                return pltpu.async_copy(tp.at[ib.at[j, 0]], pbufs[p],
                                        gsems[p])

            idesc(0, 0).start()

            def outer(k2, carry):
                for p in range(2):
                    g = 2 * k2 + p
                    idesc(g, p).wait()
                    gnxt = jnp.minimum(g + 1, ngroups - 1)
                    idesc(gnxt, 1 - p).start()
                    ib = ibufs[p]
                    gd = {0: gfire(ib, 0, 0), 1: gfire(ib, 1, 1)}
                    sd = {}
                    for j in range(GROUP):
                        b = j % 2
                        gd[j].wait()
                        if j >= 1:
                            sd[j - 1].wait()
                        cvt(pbufs[b])
                        if j + 2 < GROUP:
                            gd[j + 2] = gfire(ib, j + 2, b)
                        sd[j] = pltpu.async_copy(fbuf, acc.at[ib.at[j, 1]],
                                                 ssem, add=True)
                    sd[GROUP - 1].wait()
                return carry

            lax.fori_loop(0, ngroups // 2, outer, 0)
            idesc(ngroups - 1, 0).wait()
            plsc.subcore_barrier()
            pltpu.sync_copy(acc.at[pl.ds(s * ROWS_T, ROWS_T), :],
                            out_hbm.at[c, pl.ds(s * ROWS_T, ROWS_T), :])

        if class_split:
            @pl.when(c == 0)
            def _():
                run(tf_a, tp_a)

            @pl.when(c != 0)
            def _():
                run(tf_b, tp_b)
        else:
            run(tf_a, tp_a)

    return functools.partial(
        pl.kernel,
        out_type=jax.ShapeDtypeStruct((NSC, NP, H), jnp.float32),
        mesh=_mesh(),
        compiler_params=pltpu.CompilerParams(use_tc_tiling_on_sc=False,
                                             needs_layout_passes=False),
        scratch_types=[
            pltpu.VMEM((GROUP, 2, 128), jnp.int32),
            pltpu.VMEM((GROUP, 2, 128), jnp.int32),
            pltpu.VMEM((K, H // 2), jnp.int32),
            pltpu.VMEM((K, H // 2), jnp.int32),
            pltpu.VMEM((K, H), jnp.float32),
            pltpu.SemaphoreType.DMA,
            pltpu.SemaphoreType.DMA,
            pltpu.SemaphoreType.DMA,
            pltpu.SemaphoreType.DMA,
            pltpu.SemaphoreType.DMA,
            pltpu.VMEM_SHARED((NP, H), jnp.float32),
        ],
    )(body)


def _pack_tab16(sf):
    q = jnp.clip(jnp.round(sf * 65535.0), 0.0, 65535.0).astype(jnp.int32)
    return q[..., :64] | (q[..., 64:] << 16)


def _pack_tab16c(sf):
    q = jnp.clip(jnp.round((sf + 8.0) * 4096.0), 0.0, 65535.0).astype(jnp.int32)
    return q[..., :64] | (q[..., 64:] << 16)


_apass_shared_pk16c = _make_apass_pk(class_split=False, unpack_kind='u16c')
_apass_class_pk16 = _make_apass_pk(class_split=True, unpack_kind='u16')

_apass_class = _make_apass(class_split=True)


def _prescale_body(x_ref, d0_ref, d1_ref, dinv_ref, s0_ref):
    deg = d0_ref[...] + d1_ref[...] + 1.0
    dinv = 1.0 / jnp.sqrt(deg)
    dinv_ref[...] = dinv
    s0_ref[...] = x_ref[...] * dinv


def _prescale(xp, d0, d1):
    return pl.pallas_call(
        _prescale_body,
        grid=(NB,),
        in_specs=[
            pl.BlockSpec((RB, H), lambda i: (i, 0)),
            pl.BlockSpec((RB, 1), lambda i: (i, 0)),
            pl.BlockSpec((RB, 1), lambda i: (i, 0)),
        ],
        out_specs=[
            pl.BlockSpec((RB, 1), lambda i: (i, 0)),
            pl.BlockSpec((RB, H), lambda i: (i, 0)),
        ],
        out_shape=[
            jax.ShapeDtypeStruct((NP, 1), jnp.float32),
            jax.ShapeDtypeStruct((NP, H), jnp.float32),
        ],
    )(xp, d0, d1)


def _l2relu(q):
    r2 = jnp.sum(q * q, axis=1, keepdims=True)
    nrm = jnp.maximum(jnp.sqrt(r2), 1e-12)
    return jnp.maximum(q / nrm, 0.0)


def _dense1_body(ua_ref, ub_ref, s0_ref, dinv_ref, w_ref, b_ref, out_ref):
    dinv = dinv_ref[...]
    t = (ua_ref[0] + ub_ref[0] - s0_ref[...]) * dinv
    q = jnp.dot(t, w_ref[0], preferred_element_type=jnp.float32) + b_ref[0]
    out_ref[0] = _l2relu(q) * dinv


def _dense1(u1p, s0, dinv, w, b):
    return pl.pallas_call(
        _dense1_body,
        grid=(C, NB),
        in_specs=[
            pl.BlockSpec((1, RB, H), lambda c, i: (0, i, 0)),
            pl.BlockSpec((1, RB, H), lambda c, i: (1, i, 0)),
            pl.BlockSpec((RB, H), lambda c, i: (i, 0)),
            pl.BlockSpec((RB, 1), lambda c, i: (i, 0)),
            pl.BlockSpec((1, H, H), lambda c, i: (c, 0, 0)),
            pl.BlockSpec((1, 1, H), lambda c, i: (c, 0, 0)),
        ],
        out_specs=pl.BlockSpec((1, RB, H), lambda c, i: (c, i, 0)),
        out_shape=jax.ShapeDtypeStruct((C, NP, H), jnp.float32),
    )(u1p, u1p, s0, dinv, w, b)


def _make_dense23(prescale_out):
    def body(u_ref, dinv_ref, w_ref, b_ref, out_ref):
        dinv = dinv_ref[...]
        t = u_ref[0] * dinv
        q = jnp.dot(t, w_ref[0], preferred_element_type=jnp.float32) + b_ref[0]
        h = _l2relu(q)
        out_ref[0] = h * dinv if prescale_out else h

    def call(u, dinv, w, b):
        return pl.pallas_call(
            body,
            grid=(C, NB),
            in_specs=[
                pl.BlockSpec((1, RB, H), lambda c, i: (c, i, 0)),
                pl.BlockSpec((RB, 1), lambda c, i: (i, 0)),
                pl.BlockSpec((1, H, H), lambda c, i: (c, 0, 0)),
                pl.BlockSpec((1, 1, H), lambda c, i: (c, 0, 0)),
            ],
            out_specs=pl.BlockSpec((1, RB, H), lambda c, i: (c, i, 0)),
            out_shape=jax.ShapeDtypeStruct((C, NP, H), jnp.float32),
        )(u, dinv, w, b)

    return call


_dense2 = _make_dense23(prescale_out=True)
_dense3 = _make_dense23(prescale_out=False)


def _pool_body(h_ref, batch_ref, w1_ref, b1_ref, w2_ref, b2_ref, out_ref):
    hb = h_ref[0]
    bt = batch_ref[0]
    gids = lax.broadcasted_iota(jnp.int32, (G, NP), 0)
    oh = (bt == gids).astype(jnp.float32)
    pooled = jnp.dot(oh, hb, preferred_element_type=jnp.float32)
    z = jnp.dot(pooled, w1_ref[0], preferred_element_type=jnp.float32)
    z = jnp.maximum(z + b1_ref[0], 0.0)
    o = jnp.sum(z * w2_ref[0], axis=1) + b2_ref[0, 0]
    out_ref[0, 0] = o


def _pool_head(h3, batchp, w1, b1, w2t, b2):
    return pl.pallas_call(
        _pool_body,
        grid=(C,),
        in_specs=[
            pl.BlockSpec((1, NP, H), lambda c: (c, 0, 0)),
            pl.BlockSpec((1, 1, NP), lambda c: (0, 0, 0)),
            pl.BlockSpec((1, H, H), lambda c: (c, 0, 0)),
            pl.BlockSpec((1, 1, H), lambda c: (c, 0, 0)),
            pl.BlockSpec((1, 1, H), lambda c: (c, 0, 0)),
            pl.BlockSpec((1, 1, 1), lambda c: (c, 0, 0)),
        ],
        out_specs=pl.BlockSpec((1, 1, G), lambda c: (c, 0, 0)),
        out_shape=jax.ShapeDtypeStruct((C, 1, G), jnp.float32),
    )(h3, batchp, w1, b1, w2t, b2)


def kernel(x, edge_index, batch, conv_W0, conv_b0, conv_W1, conv_b1,
           conv_W2, conv_b2, lin1_W, lin1_b, lin2_W, lin2_b):
    pad_e = EP - E
    srcp = jnp.concatenate(
        [edge_index[0], jnp.zeros((pad_e,), edge_index.dtype)]).reshape(ER, 128)
    dstp = jnp.concatenate(
        [edge_index[1], jnp.full((pad_e,), N, edge_index.dtype)]).reshape(ER, 128)
    idx2 = jnp.stack([srcp, dstp], axis=1)
    xp = jnp.pad(x, ((0, NP - N), (0, 0)))
    batchp = jnp.pad(batch, (0, NP - N), constant_values=G).reshape(1, 1, NP)

    degp = _deg_pass(idx2)
    d0 = degp[0].reshape(NP, 1)
    d1 = degp[1].reshape(NP, 1)
    dinv, s0 = _prescale(xp, d0, d1)

    b0 = conv_b0.reshape(C, 1, H)
    b1 = conv_b1.reshape(C, 1, H)
    b2 = conv_b2.reshape(C, 1, H)
    l1b = lin1_b.reshape(C, 1, H)
    w2t = jnp.transpose(lin2_W, (0, 2, 1))
    l2b = lin2_b.reshape(C, 1, 1)

    u1p = _apass_shared_pk16c(s0, _pack_tab16c(s0), idx2)
    s1 = _dense1(u1p, s0, dinv, conv_W0, b0)
    s1p = _pack_tab16(s1)
    u2 = _apass_class_pk16(s1[0], s1[1], s1p[0], s1p[1], idx2)
    s2 = _dense2(u2, dinv, conv_W1, b1)
    s2p = _pack_tab16(s2)
    u3 = _apass_class_pk16(s2[0], s2[1], s2p[0], s2p[1], idx2)
    h3 = _dense3(u3, dinv, conv_W2, b2)
    out = _pool_head(h3, batchp, lin1_W, l1b, w2t, l2b)
    return jnp.transpose(out[:, 0, :], (1, 0))
